# baseline (speedup 1.0000x reference)
import functools

import jax
import jax.numpy as jnp
from jax.experimental import pallas as pl

_THRESHOLD = 0.2
_TEMPERATURE = 1.0


def _fused_body(B, E, R, x_ref, w_ref, b_ref, v_ref,
                o_ref, coeffs_ref, mask_ref):
    o_ref[...] = (
        jnp.dot(x_ref[...], w_ref[...], preferred_element_type=jnp.float32)
        + b_ref[...]
    )

    @pl.when(pl.program_id(0) == 0)
    def _routing():
        tok = x_ref[:B, :]
        nsq_cols = []
        for e in range(E):
            proj_e = jnp.dot(tok, v_ref[e],
                             preferred_element_type=jnp.float32)
            recon_e = jax.lax.dot_general(
                proj_e, v_ref[e],
                dimension_numbers=(((1,), (1,)), ((), ())),
                preferred_element_type=jnp.float32)
            resid_e = tok - recon_e
            nsq_cols.append(jnp.sum(resid_e * resid_e, axis=1, keepdims=True))
        nsq = jnp.concatenate(nsq_cols, axis=1)
        logits = -jnp.sqrt(nsq + 1e-12)
        mean = jnp.mean(logits, axis=1, keepdims=True)
        var = jnp.sum((logits - mean) ** 2, axis=1, keepdims=True) / (E - 1)
        std = jnp.sqrt(var) + 1e-06
        z = (logits - mean) / std / _TEMPERATURE
        zmax = jnp.max(z, axis=1, keepdims=True)
        ez = jnp.exp(z - zmax)
        coeffs_ref[...] = ez / jnp.sum(ez, axis=1, keepdims=True)
        mask_ref[...] = (coeffs_ref[...] > _THRESHOLD).astype(jnp.int8)


@functools.partial(jax.jit, static_argnames=("bm",))
def _run(x, v, s, W, b, bm=3152):
    SEQ, B, D = x.shape
    E, _, R = v.shape
    M = SEQ * B
    xm = x.reshape(M, D)
    Wt = W.T
    b2 = b.reshape(1, D)
    grid = pl.cdiv(M, bm)
    out, coeffs, mask_i8 = pl.pallas_call(
        functools.partial(_fused_body, B, E, R),
        grid=(grid,),
        in_specs=[
            pl.BlockSpec((bm, D), lambda i: (i, 0)),
            pl.BlockSpec((D, D), lambda i: (0, 0)),
            pl.BlockSpec((1, D), lambda i: (0, 0)),
            pl.BlockSpec((E, D, R), lambda i: (0, 0, 0)),
        ],
        out_specs=[
            pl.BlockSpec((bm, D), lambda i: (i, 0)),
            pl.BlockSpec((B, E), lambda i: (0, 0)),
            pl.BlockSpec((B, E), lambda i: (0, 0)),
        ],
        out_shape=[
            jax.ShapeDtypeStruct((M, D), jnp.float32),
            jax.ShapeDtypeStruct((B, E), jnp.float32),
            jax.ShapeDtypeStruct((B, E), jnp.int8),
        ],
    )(xm, Wt, b2, v)
    return out.reshape(SEQ, B, D), coeffs, mask_i8.astype(jnp.bool_)


def kernel(x, v, s, W, b, bsz):
    return _run(x, v, s, W, b)


# speedup vs baseline: 1.0159x; 1.0159x over previous
import functools

import jax
import jax.numpy as jnp
from jax.experimental import pallas as pl

_THRESHOLD = 0.2
_TEMPERATURE = 1.0


def _fused_body(B, E, R, x_ref, w_ref, b_ref, vflat_ref, vt_ref,
                o_ref, coeffs_ref, mask_ref):
    o_ref[...] = (
        jnp.dot(x_ref[...], w_ref[...], preferred_element_type=jnp.float32)
        + b_ref[...]
    )

    @pl.when(pl.program_id(0) == 0)
    def _routing():
        tok = x_ref[:B, :]
        proj = jnp.dot(tok, vflat_ref[...],
                       preferred_element_type=jnp.float32)
        nsq_cols = []
        for e in range(E):
            proj_e = proj[:, e * R:(e + 1) * R]
            recon_e = jnp.dot(proj_e, vt_ref[e],
                              preferred_element_type=jnp.float32)
            resid_e = tok - recon_e
            nsq_cols.append(jnp.sum(resid_e * resid_e, axis=1, keepdims=True))
        nsq = jnp.concatenate(nsq_cols, axis=1)
        logits = -jnp.sqrt(nsq + 1e-12)
        mean = jnp.mean(logits, axis=1, keepdims=True)
        var = jnp.sum((logits - mean) ** 2, axis=1, keepdims=True) / (E - 1)
        std = jnp.sqrt(var) + 1e-06
        z = (logits - mean) / std / _TEMPERATURE
        zmax = jnp.max(z, axis=1, keepdims=True)
        ez = jnp.exp(z - zmax)
        coeffs_ref[...] = ez / jnp.sum(ez, axis=1, keepdims=True)
        mask_ref[...] = (coeffs_ref[...] > _THRESHOLD).astype(jnp.int8)


@functools.partial(jax.jit, static_argnames=("bm"))
def _run(x, v, s, W, b, bm=3584):
    SEQ, B, D = x.shape
    E, _, R = v.shape
    M = SEQ * B
    xm = x.reshape(M, D)
    Wt = W.T
    b2 = b.reshape(1, D)
    vflat = v.transpose(1, 0, 2).reshape(D, E * R)
    vt = v.transpose(0, 2, 1)
    grid = pl.cdiv(M, bm)
    out, coeffs, mask_i8 = pl.pallas_call(
        functools.partial(_fused_body, B, E, R),
        grid=(grid,),
        in_specs=[
            pl.BlockSpec((bm, D), lambda i: (i, 0)),
            pl.BlockSpec((D, D), lambda i: (0, 0)),
            pl.BlockSpec((1, D), lambda i: (0, 0)),
            pl.BlockSpec((D, E * R), lambda i: (0, 0)),
            pl.BlockSpec((E, R, D), lambda i: (0, 0, 0)),
        ],
        out_specs=[
            pl.BlockSpec((bm, D), lambda i: (i, 0)),
            pl.BlockSpec((B, E), lambda i: (0, 0)),
            pl.BlockSpec((B, E), lambda i: (0, 0)),
        ],
        out_shape=[
            jax.ShapeDtypeStruct((M, D), jnp.float32),
            jax.ShapeDtypeStruct((B, E), jnp.float32),
            jax.ShapeDtypeStruct((B, E), jnp.int8),
        ],
    )(xm, Wt, b2, vflat, vt)
    return out.reshape(SEQ, B, D), coeffs, mask_i8.astype(jnp.bool_)


def kernel(x, v, s, W, b, bsz):
    return _run(x, v, s, W, b)


# speedup vs baseline: 1.0322x; 1.0160x over previous
import functools

import jax
import jax.numpy as jnp
from jax.experimental import pallas as pl

_THRESHOLD = 0.2
_TEMPERATURE = 1.0


def _fused_body(B, E, R, x_ref, w_ref, b_ref, vflat_ref, vt_ref,
                o_ref, coeffs_ref, mask_ref):
    o_ref[...] = (
        jnp.dot(x_ref[...].astype(jnp.bfloat16), w_ref[...],
                preferred_element_type=jnp.float32)
        + b_ref[...]
    )

    @pl.when(pl.program_id(0) == 0)
    def _routing():
        tok = x_ref[:B, :]
        proj = jnp.dot(tok, vflat_ref[...],
                       preferred_element_type=jnp.float32)
        nsq_cols = []
        for e in range(E):
            proj_e = proj[:, e * R:(e + 1) * R]
            recon_e = jnp.dot(proj_e, vt_ref[e],
                              preferred_element_type=jnp.float32)
            resid_e = tok - recon_e
            nsq_cols.append(jnp.sum(resid_e * resid_e, axis=1, keepdims=True))
        nsq = jnp.concatenate(nsq_cols, axis=1)
        logits = -jnp.sqrt(nsq + 1e-12)
        mean = jnp.mean(logits, axis=1, keepdims=True)
        var = jnp.sum((logits - mean) ** 2, axis=1, keepdims=True) / (E - 1)
        std = jnp.sqrt(var) + 1e-06
        z = (logits - mean) / std / _TEMPERATURE
        zmax = jnp.max(z, axis=1, keepdims=True)
        ez = jnp.exp(z - zmax)
        coeffs_ref[...] = ez / jnp.sum(ez, axis=1, keepdims=True)
        mask_ref[...] = (coeffs_ref[...] > _THRESHOLD).astype(jnp.int8)


@functools.partial(jax.jit, static_argnames=("bm"))
def _run(x, v, s, W, b, bm=3152):
    SEQ, B, D = x.shape
    E, _, R = v.shape
    M = SEQ * B
    xm = x.reshape(M, D)
    Wt = W.T.astype(jnp.bfloat16)
    b2 = b.reshape(1, D)
    vflat = v.transpose(1, 0, 2).reshape(D, E * R)
    vt = v.transpose(0, 2, 1)
    grid = pl.cdiv(M, bm)
    out, coeffs, mask_i8 = pl.pallas_call(
        functools.partial(_fused_body, B, E, R),
        grid=(grid,),
        in_specs=[
            pl.BlockSpec((bm, D), lambda i: (i, 0)),
            pl.BlockSpec((D, D), lambda i: (0, 0)),
            pl.BlockSpec((1, D), lambda i: (0, 0)),
            pl.BlockSpec((D, E * R), lambda i: (0, 0)),
            pl.BlockSpec((E, R, D), lambda i: (0, 0, 0)),
        ],
        out_specs=[
            pl.BlockSpec((bm, D), lambda i: (i, 0)),
            pl.BlockSpec((B, E), lambda i: (0, 0)),
            pl.BlockSpec((B, E), lambda i: (0, 0)),
        ],
        out_shape=[
            jax.ShapeDtypeStruct((M, D), jnp.float32),
            jax.ShapeDtypeStruct((B, E), jnp.float32),
            jax.ShapeDtypeStruct((B, E), jnp.int8),
        ],
    )(xm, Wt, b2, vflat, vt)
    return out.reshape(SEQ, B, D), coeffs, mask_i8.astype(jnp.bool_)


def kernel(x, v, s, W, b, bsz):
    return _run(x, v, s, W, b)


# speedup vs baseline: 1.0329x; 1.0006x over previous
import functools

import jax
import jax.numpy as jnp
from jax.experimental import pallas as pl
from jax.experimental.pallas import tpu as pltpu

_THRESHOLD = 0.2
_TEMPERATURE = 1.0


def _fused_body(B, E, R, x_ref, w_ref, b_ref, vflat_ref, vt_ref,
                o_ref, coeffs_ref, mask_ref):
    o_ref[...] = (
        jnp.dot(x_ref[...].astype(jnp.bfloat16), w_ref[...],
                preferred_element_type=jnp.float32)
        + b_ref[...]
    )

    @pl.when(pl.program_id(0) == 0)
    def _routing():
        tok = x_ref[:B, :]
        proj = jnp.dot(tok, vflat_ref[...],
                       preferred_element_type=jnp.float32)
        nsq_cols = []
        for e in range(E):
            proj_e = proj[:, e * R:(e + 1) * R]
            recon_e = jnp.dot(proj_e, vt_ref[e],
                              preferred_element_type=jnp.float32)
            resid_e = tok - recon_e
            nsq_cols.append(jnp.sum(resid_e * resid_e, axis=1, keepdims=True))
        nsq = jnp.concatenate(nsq_cols, axis=1)
        logits = -jnp.sqrt(nsq + 1e-12)
        mean = jnp.mean(logits, axis=1, keepdims=True)
        var = jnp.sum((logits - mean) ** 2, axis=1, keepdims=True) / (E - 1)
        std = jnp.sqrt(var) + 1e-06
        z = (logits - mean) / std / _TEMPERATURE
        zmax = jnp.max(z, axis=1, keepdims=True)
        ez = jnp.exp(z - zmax)
        coeffs_ref[...] = ez / jnp.sum(ez, axis=1, keepdims=True)
        mask_ref[...] = (coeffs_ref[...] > _THRESHOLD).astype(jnp.int8)


@functools.partial(jax.jit, static_argnames=("bm"))
def _run(x, v, s, W, b, bm=3152):
    SEQ, B, D = x.shape
    E, _, R = v.shape
    M = SEQ * B
    xm = x.reshape(M, D)
    Wt = W.T.astype(jnp.bfloat16)
    b2 = b.reshape(1, D)
    vflat = v.transpose(1, 0, 2).reshape(D, E * R)
    vt = v.transpose(0, 2, 1)
    grid = pl.cdiv(M, bm)
    out, coeffs, mask_i8 = pl.pallas_call(
        functools.partial(_fused_body, B, E, R),
        grid=(grid,),
        in_specs=[
            pl.BlockSpec((bm, D), lambda i: (i, 0)),
            pl.BlockSpec((D, D), lambda i: (0, 0)),
            pl.BlockSpec((1, D), lambda i: (0, 0)),
            pl.BlockSpec((D, E * R), lambda i: (0, 0)),
            pl.BlockSpec((E, R, D), lambda i: (0, 0, 0)),
        ],
        out_specs=[
            pl.BlockSpec((bm, D), lambda i: (i, 0)),
            pl.BlockSpec((B, E), lambda i: (0, 0)),
            pl.BlockSpec((B, E), lambda i: (0, 0)),
        ],
        out_shape=[
            jax.ShapeDtypeStruct((M, D), jnp.float32),
            jax.ShapeDtypeStruct((B, E), jnp.float32),
            jax.ShapeDtypeStruct((B, E), jnp.int8),
        ],
        compiler_params=pltpu.CompilerParams(
            dimension_semantics=("parallel",)),
    )(xm, Wt, b2, vflat, vt)
    return out.reshape(SEQ, B, D), coeffs, mask_i8.astype(jnp.bool_)


def kernel(x, v, s, W, b, bsz):
    return _run(x, v, s, W, b)
